# baseline (speedup 1.0000x reference)
import functools

import jax
import jax.numpy as jnp
from jax.experimental import pallas as pl
from jax.experimental.pallas import tpu as pltpu


def _nbeats_kernel(x_hbm, w1_ref, b1_ref, w2_ref, b2_ref, w3t_ref, b3_ref,
                   o_ref, xbuf, sem, *, nb, nchunks):
    cb = x_hbm.shape[0] // nchunks

    def chunk_copy(i, slot):
        return pltpu.make_async_copy(
            x_hbm.at[pl.ds(i * cb, cb), :], xbuf.at[slot], sem.at[slot])

    chunk_copy(0, 0).start()

    w1b = [w1_ref[b].astype(jnp.bfloat16) for b in range(nb)]
    w2b = [w2_ref[b].astype(jnp.bfloat16) for b in range(nb)]
    w3b = [w3t_ref[b].astype(jnp.bfloat16) for b in range(nb)]
    b3s = jnp.sum(b3_ref[...], axis=0, keepdims=True)
    b3col = jnp.swapaxes(b3s, 0, 1)

    for i in range(nchunks):
        if i + 1 < nchunks:
            chunk_copy(i + 1, (i + 1) % 2).start()
        chunk_copy(i, i % 2).wait()
        inp = xbuf[i % 2]
        out_t = b3col
        for b in range(nb):
            h = jnp.dot(inp, w1b[b], preferred_element_type=jnp.float32)
            h = jnp.maximum(h + b1_ref[b:b + 1, :], 0.0).astype(jnp.bfloat16)
            h = jnp.dot(h, w2b[b], preferred_element_type=jnp.float32)
            h = jnp.maximum(h + b2_ref[b:b + 1, :], 0.0).astype(jnp.bfloat16)
            out_t = out_t + jnp.einsum("mk,nk->mn", w3b[b], h,
                                       preferred_element_type=jnp.float32)
        o_ref[:, i * cb:(i + 1) * cb] = out_t


def kernel(x, w1, b1, w2, b2, w3, b3):
    B, t_in, nf = x.shape
    nb, _, hid = w1.shape
    t_out = w3.shape[-1]

    inp = x[:, :, -1].astype(jnp.bfloat16)
    w3t = jnp.swapaxes(w3, 1, 2)

    nchunks = 8 if B % (8 * 8) == 0 else 1
    cb = B // nchunks
    out_t = pl.pallas_call(
        functools.partial(_nbeats_kernel, nb=nb, nchunks=nchunks),
        out_shape=jax.ShapeDtypeStruct((t_out, B), jnp.float32),
        in_specs=[
            pl.BlockSpec(memory_space=pl.ANY),
            pl.BlockSpec(w1.shape, lambda: (0, 0, 0)),
            pl.BlockSpec(b1.shape, lambda: (0, 0)),
            pl.BlockSpec(w2.shape, lambda: (0, 0, 0)),
            pl.BlockSpec(b2.shape, lambda: (0, 0)),
            pl.BlockSpec((nb, t_out, hid), lambda: (0, 0, 0)),
            pl.BlockSpec(b3.shape, lambda: (0, 0)),
        ],
        out_specs=pl.BlockSpec((t_out, B), lambda: (0, 0)),
        scratch_shapes=[
            pltpu.VMEM((2, cb, t_in), jnp.bfloat16),
            pltpu.SemaphoreType.DMA((2,)),
        ],
    )(inp, w1, b1, w2, b2, w3t, b3)
    return out_t.T


# speedup vs baseline: 1.0884x; 1.0884x over previous
import functools

import jax
import jax.numpy as jnp
from jax.experimental import pallas as pl
from jax.experimental.pallas import tpu as pltpu


def _nbeats_kernel(x_hbm, w1_ref, b1_ref, w2_ref, b2_ref, w3t_ref, b3_ref,
                   o_ref, xbuf, sem, *, nb, nchunks):
    cb = x_hbm.shape[0] // nchunks

    def chunk_copy(i, slot):
        return pltpu.make_async_copy(
            x_hbm.at[pl.ds(i * cb, cb), :], xbuf.at[slot], sem.at[slot])

    chunk_copy(0, 0).start()

    w1b = [w1_ref[b].astype(jnp.bfloat16) for b in range(nb)]
    w2b = [w2_ref[b].astype(jnp.bfloat16) for b in range(nb)]
    w3b = [w3t_ref[b].astype(jnp.bfloat16) for b in range(nb)]
    b3s = jnp.sum(b3_ref[...], axis=0, keepdims=True)
    b3col = jnp.swapaxes(b3s, 0, 1)

    for i in range(nchunks):
        if i + 1 < nchunks:
            chunk_copy(i + 1, (i + 1) % 2).start()
        chunk_copy(i, i % 2).wait()
        inp = xbuf[i % 2]
        out_t = b3col
        for b in range(nb):
            h = jnp.dot(inp, w1b[b], preferred_element_type=jnp.float32)
            h = jnp.maximum(h + b1_ref[b:b + 1, :], 0.0).astype(jnp.bfloat16)
            h = jnp.dot(h, w2b[b], preferred_element_type=jnp.float32)
            h = jnp.maximum(h + b2_ref[b:b + 1, :], 0.0).astype(jnp.bfloat16)
            out_t = out_t + jnp.einsum("mk,nk->mn", w3b[b], h,
                                       preferred_element_type=jnp.float32)
        o_ref[:, i * cb:(i + 1) * cb] = out_t


def kernel(x, w1, b1, w2, b2, w3, b3):
    B, t_in, nf = x.shape
    nb, _, hid = w1.shape
    t_out = w3.shape[-1]

    inp = x[:, :, -1].astype(jnp.bfloat16)
    w3t = jnp.swapaxes(w3, 1, 2)

    nchunks = 4 if B % (4 * 8) == 0 else 1
    cb = B // nchunks
    out_t = pl.pallas_call(
        functools.partial(_nbeats_kernel, nb=nb, nchunks=nchunks),
        out_shape=jax.ShapeDtypeStruct((t_out, B), jnp.float32),
        in_specs=[
            pl.BlockSpec(memory_space=pl.ANY),
            pl.BlockSpec(w1.shape, lambda: (0, 0, 0)),
            pl.BlockSpec(b1.shape, lambda: (0, 0)),
            pl.BlockSpec(w2.shape, lambda: (0, 0, 0)),
            pl.BlockSpec(b2.shape, lambda: (0, 0)),
            pl.BlockSpec((nb, t_out, hid), lambda: (0, 0, 0)),
            pl.BlockSpec(b3.shape, lambda: (0, 0)),
        ],
        out_specs=pl.BlockSpec((t_out, B), lambda: (0, 0)),
        scratch_shapes=[
            pltpu.VMEM((2, cb, t_in), jnp.bfloat16),
            pltpu.SemaphoreType.DMA((2,)),
        ],
    )(inp, w1, b1, w2, b2, w3t, b3)
    return out_t.T


# speedup vs baseline: 1.2446x; 1.1435x over previous
import functools

import jax
import jax.numpy as jnp
from jax.experimental import pallas as pl
from jax.experimental.pallas import tpu as pltpu


def _nbeats_kernel(x_ref, w1_ref, b1_ref, w2_ref, b2_ref, w3t_ref, b3_ref,
                   o_ref, *, nb):
    inp = x_ref[...]
    b3s = jnp.sum(b3_ref[...], axis=0, keepdims=True)
    out_t = jnp.swapaxes(b3s, 0, 1)
    hid = w1_ref.shape[-1]
    w1cat = jnp.concatenate(
        [w1_ref[b].astype(jnp.bfloat16) for b in range(nb)], axis=1)
    b1cat = jnp.concatenate([b1_ref[b:b + 1, :] for b in range(nb)], axis=1)
    h1 = jnp.dot(inp, w1cat, preferred_element_type=jnp.float32)
    h1 = jnp.maximum(h1 + b1cat, 0.0).astype(jnp.bfloat16)
    for b in range(nb):
        h = jnp.dot(h1[:, b * hid:(b + 1) * hid],
                    w2_ref[b].astype(jnp.bfloat16),
                    preferred_element_type=jnp.float32)
        h = jnp.maximum(h + b2_ref[b:b + 1, :], 0.0).astype(jnp.bfloat16)
        out_t = out_t + jnp.einsum("mk,nk->mn",
                                   w3t_ref[b].astype(jnp.bfloat16), h,
                                   preferred_element_type=jnp.float32)
    o_ref[...] = out_t


def kernel(x, w1, b1, w2, b2, w3, b3):
    B, t_in, nf = x.shape
    nb, _, hid = w1.shape
    t_out = w3.shape[-1]

    inp = x[:, :, -1].astype(jnp.bfloat16)
    w3t = jnp.swapaxes(w3, 1, 2)

    tb = B
    out_t = pl.pallas_call(
        functools.partial(_nbeats_kernel, nb=nb),
        out_shape=jax.ShapeDtypeStruct((t_out, B), jnp.float32),
        grid=(B // tb,),
        in_specs=[
            pl.BlockSpec((tb, t_in), lambda i: (i, 0)),
            pl.BlockSpec(w1.shape, lambda i: (0, 0, 0)),
            pl.BlockSpec(b1.shape, lambda i: (0, 0)),
            pl.BlockSpec(w2.shape, lambda i: (0, 0, 0)),
            pl.BlockSpec(b2.shape, lambda i: (0, 0)),
            pl.BlockSpec((nb, t_out, hid), lambda i: (0, 0, 0)),
            pl.BlockSpec(b3.shape, lambda i: (0, 0)),
        ],
        out_specs=pl.BlockSpec((t_out, tb), lambda i: (0, i)),
        compiler_params=pltpu.CompilerParams(
            dimension_semantics=("parallel",)),
    )(inp, w1, b1, w2, b2, w3t, b3)
    return out_t.T


# speedup vs baseline: 1.2628x; 1.0147x over previous
import functools

import jax
import jax.numpy as jnp
from jax.experimental import pallas as pl
from jax.experimental.pallas import tpu as pltpu


def _nbeats_kernel(x_ref, w1_ref, b1_ref, w2_ref, b2_ref, w3t_ref, b3_ref,
                   o_ref, *, nb):
    inp = x_ref[...]
    b3s = jnp.sum(b3_ref[...], axis=0, keepdims=True)
    out_t = jnp.swapaxes(b3s, 0, 1)
    hid = w1_ref.shape[-1]
    w1cat = jnp.concatenate(
        [w1_ref[b].astype(jnp.bfloat16) for b in range(nb)], axis=1)
    b1cat = jnp.concatenate([b1_ref[b:b + 1, :] for b in range(nb)], axis=1)
    h1 = jnp.dot(inp, w1cat, preferred_element_type=jnp.float32)
    h1 = jnp.maximum(h1 + b1cat, 0.0).astype(jnp.bfloat16)
    hs = []
    for b in range(nb):
        h = jnp.dot(h1[:, b * hid:(b + 1) * hid],
                    w2_ref[b].astype(jnp.bfloat16),
                    preferred_element_type=jnp.float32)
        hs.append(jnp.maximum(h + b2_ref[b:b + 1, :], 0.0).astype(jnp.bfloat16))
    hcat = jnp.concatenate(hs, axis=1)
    w3cat = jnp.concatenate(
        [w3t_ref[b].astype(jnp.bfloat16) for b in range(nb)], axis=1)
    out_t = out_t + jnp.einsum("mk,nk->mn", w3cat, hcat,
                               preferred_element_type=jnp.float32)
    o_ref[...] = out_t


def kernel(x, w1, b1, w2, b2, w3, b3):
    B, t_in, nf = x.shape
    nb, _, hid = w1.shape
    t_out = w3.shape[-1]

    inp = x[:, :, -1].astype(jnp.bfloat16)
    w3t = jnp.swapaxes(w3, 1, 2)

    tb = B
    out_t = pl.pallas_call(
        functools.partial(_nbeats_kernel, nb=nb),
        out_shape=jax.ShapeDtypeStruct((t_out, B), jnp.float32),
        grid=(B // tb,),
        in_specs=[
            pl.BlockSpec((tb, t_in), lambda i: (i, 0)),
            pl.BlockSpec(w1.shape, lambda i: (0, 0, 0)),
            pl.BlockSpec(b1.shape, lambda i: (0, 0)),
            pl.BlockSpec(w2.shape, lambda i: (0, 0, 0)),
            pl.BlockSpec(b2.shape, lambda i: (0, 0)),
            pl.BlockSpec((nb, t_out, hid), lambda i: (0, 0, 0)),
            pl.BlockSpec(b3.shape, lambda i: (0, 0)),
        ],
        out_specs=pl.BlockSpec((t_out, tb), lambda i: (0, i)),
        compiler_params=pltpu.CompilerParams(
            dimension_semantics=("parallel",)),
    )(inp, w1, b1, w2, b2, w3t, b3)
    return out_t.T


# speedup vs baseline: 1.2735x; 1.0085x over previous
import functools

import jax
import jax.numpy as jnp
from jax.experimental import pallas as pl
from jax.experimental.pallas import tpu as pltpu


def _nbeats_kernel(x_ref, w1_ref, b1_ref, w2_ref, b2_ref, w3t_ref, b3_ref,
                   o_ref, *, nb):
    inp = x_ref[...]
    b3s = jnp.sum(b3_ref[...], axis=0, keepdims=True)
    out_t = jnp.swapaxes(b3s, 0, 1)
    hid = w1_ref.shape[-1]
    w1cat = jnp.concatenate(
        [w1_ref[b].astype(jnp.bfloat16) for b in range(nb)], axis=1)
    b1cat = jnp.concatenate([b1_ref[b:b + 1, :] for b in range(nb)], axis=1)
    h1 = jnp.dot(inp, w1cat, preferred_element_type=jnp.float32)
    h1 = jnp.maximum(h1 + b1cat, 0.0).astype(jnp.bfloat16)
    hs = []
    for b in range(nb):
        h = jnp.dot(h1[:, b * hid:(b + 1) * hid],
                    w2_ref[b].astype(jnp.bfloat16),
                    preferred_element_type=jnp.float32)
        hs.append(jnp.maximum(h + b2_ref[b:b + 1, :], 0.0).astype(jnp.bfloat16))
    hcat = jnp.concatenate(hs, axis=1)
    w3cat = jnp.concatenate(
        [w3t_ref[b].astype(jnp.bfloat16) for b in range(nb)], axis=1)
    out_t = out_t + jnp.einsum("mk,nk->mn", w3cat, hcat,
                               preferred_element_type=jnp.float32)
    o_ref[...] = out_t


def kernel(x, w1, b1, w2, b2, w3, b3):
    B, t_in, nf = x.shape
    nb, _, hid = w1.shape
    t_out = w3.shape[-1]

    inp = x[:, :, -1].astype(jnp.bfloat16)
    w3t = jnp.swapaxes(w3, 1, 2)

    tb = 2048 if B % 2048 == 0 else B
    out_t = pl.pallas_call(
        functools.partial(_nbeats_kernel, nb=nb),
        out_shape=jax.ShapeDtypeStruct((t_out, B), jnp.float32),
        grid=(B // tb,),
        in_specs=[
            pl.BlockSpec((tb, t_in), lambda i: (i, 0)),
            pl.BlockSpec(w1.shape, lambda i: (0, 0, 0)),
            pl.BlockSpec(b1.shape, lambda i: (0, 0)),
            pl.BlockSpec(w2.shape, lambda i: (0, 0, 0)),
            pl.BlockSpec(b2.shape, lambda i: (0, 0)),
            pl.BlockSpec((nb, t_out, hid), lambda i: (0, 0, 0)),
            pl.BlockSpec(b3.shape, lambda i: (0, 0)),
        ],
        out_specs=pl.BlockSpec((t_out, tb), lambda i: (0, i)),
        compiler_params=pltpu.CompilerParams(
            dimension_semantics=("parallel",)),
    )(inp, w1, b1, w2, b2, w3t, b3)
    return out_t.T
